# Initial kernel scaffold; baseline (speedup 1.0000x reference)
import jax
import jax.numpy as jnp
from jax.experimental import pallas as pl


def kernel(input_labels, out_labels, negatives, table):
    raise NotImplementedError("write your pallas kernel here")


# speedup vs baseline: 1.7422x; 1.7422x over previous
import functools

import jax
import jax.numpy as jnp
from jax import lax
from jax.experimental import pallas as pl
from jax.experimental.pallas import tpu as pltpu
from jax.experimental.pallas import tpu_sc as plsc

_NC = 2
_NS = 16
_NW = _NC * _NS
_L = 16
_NEG = 5
_NIDX = _NEG + 2
_CHUNK = 64


def _lane_shuffle(x, idx):
    return lax.gather(
        x, idx[:, None],
        lax.GatherDimensionNumbers(
            offset_dims=(), collapsed_slice_dims=(0,), start_index_map=(0,)),
        (1,),
        mode=lax.GatherScatterMode.PROMISE_IN_BOUNDS,
    )


def _sc_body(nch, table_hbm, idx_hbm, out_hbm, idx_v, rows_v, scores_v, sem):
    wid = lax.axis_index("s") * _NC + lax.axis_index("c")
    pltpu.sync_copy(idx_hbm.at[wid], idx_v)
    lane = lax.iota(jnp.int32, _L)
    perms = [jnp.bitwise_xor(lane, s) for s in (8, 4, 2, 1)]

    def chunk_body(c, carry):
        cps = [
            pltpu.async_copy(table_hbm.at[idx_v.at[r, c]], rows_v.at[r], sem)
            for r in range(_NIDX)
        ]
        for cp in cps:
            cp.wait()

        def grp_body(g, gc):
            b0 = g * _L
            res = [None] * 6
            for i in range(_L):
                b = b0 + i
                inp = [rows_v[0, b, pl.ds(_L * k, _L)] for k in range(8)]
                for j in range(6):
                    acc = inp[0] * rows_v[j + 1, b, pl.ds(0, _L)]
                    for k in range(1, 8):
                        acc = acc + inp[k] * rows_v[j + 1, b, pl.ds(_L * k, _L)]
                    if j > 0:
                        acc = 0.0 - acc
                    for p in perms:
                        acc = acc + _lane_shuffle(acc, p)
                    res[j] = acc if i == 0 else jnp.where(lane == i, acc, res[j])
            for j in range(6):
                scores_v[j, pl.ds(b0, _L)] = res[j]
            return gc

        lax.fori_loop(0, _CHUNK // _L, grp_body, 0, unroll=False)
        pltpu.sync_copy(scores_v, out_hbm.at[wid * nch + c])
        return carry

    lax.fori_loop(0, nch, chunk_body, 0, unroll=False)


def _sc_scores(table, idx, batch, nch):
    mesh = plsc.VectorSubcoreMesh(
        core_axis_name="c", subcore_axis_name="s",
        num_cores=_NC, num_subcores=_NS,
    )
    fn = pl.kernel(
        functools.partial(_sc_body, nch),
        out_type=jax.ShapeDtypeStruct((_NW * nch, 6, _CHUNK), jnp.float32),
        mesh=mesh,
        scratch_types=[
            pltpu.VMEM((_NIDX, nch, _CHUNK), jnp.int32),
            pltpu.VMEM((_NIDX, _CHUNK, 128), jnp.float32),
            pltpu.VMEM((6, _CHUNK), jnp.float32),
            pltpu.SemaphoreType.DMA,
        ],
    )
    return fn(table, idx)


def _tc_reduce_body(batch, x_ref, o_ref):
    x = x_ref[...]
    ls = jnp.minimum(x, 0.0) - jnp.log(1.0 + jnp.exp(-jnp.abs(x)))
    o_ref[0, 0] = -jnp.sum(ls) / batch


def kernel(input_labels, out_labels, negatives, table):
    batch = input_labels.shape[0]
    assert batch % (_NW * _CHUNK) == 0
    nch = batch // (_NW * _CHUNK)

    il = input_labels.astype(jnp.int32)
    ol = out_labels.astype(jnp.int32)
    ng = negatives.astype(jnp.int32)
    idx7 = jnp.concatenate([il[None, :], ol[None, :], ng.T], axis=0)
    idx = idx7.reshape(_NIDX, _NW, nch, _CHUNK).transpose(1, 0, 2, 3)

    scores = _sc_scores(table.astype(jnp.float32), idx, batch, nch)

    loss = pl.pallas_call(
        functools.partial(_tc_reduce_body, batch),
        out_shape=jax.ShapeDtypeStruct((1, 1), jnp.float32),
        out_specs=pl.BlockSpec(memory_space=pltpu.SMEM),
    )(scores.reshape(batch * 6 // 128, 128))
    return loss[0, 0]


# speedup vs baseline: 2.6770x; 1.5365x over previous
import functools

import jax
import jax.numpy as jnp
from jax import lax
from jax.experimental import pallas as pl
from jax.experimental.pallas import tpu as pltpu
from jax.experimental.pallas import tpu_sc as plsc

_NC = 2
_NS = 16
_NW = _NC * _NS
_L = 16
_NEG = 5
_NIDX = _NEG + 2
_CHUNK = 32


def _lane_shuffle(x, idx):
    return lax.gather(
        x, idx[:, None],
        lax.GatherDimensionNumbers(
            offset_dims=(), collapsed_slice_dims=(0,), start_index_map=(0,)),
        (1,),
        mode=lax.GatherScatterMode.PROMISE_IN_BOUNDS,
    )


def _fold16(vs, lane, perms):
    cur = list(vs)
    for s, p in zip((8, 4, 2, 1), perms):
        h = len(cur) // 2
        nxt = []
        for i in range(h):
            a = cur[i] + _lane_shuffle(cur[i], p)
            b = cur[i + h] + _lane_shuffle(cur[i + h], p)
            nxt.append(jnp.where((lane & s) == 0, a, b))
        cur = nxt
    return cur[0]


def _sc_body(nch, table_hbm, idx_hbm, out_hbm,
             idx_v, rows_v, acc_v, scores_v, sem0, sem1):
    wid = lax.axis_index("s") * _NC + lax.axis_index("c")
    pltpu.sync_copy(idx_hbm.at[wid], idx_v)
    lane = lax.iota(jnp.int32, _L)
    perms = [jnp.bitwise_xor(lane, s) for s in (8, 4, 2, 1)]

    def fire(c, par, sem):
        for r in range(_NIDX):
            pltpu.async_copy(table_hbm.at[idx_v.at[r, c]],
                             rows_v.at[par, r], sem)

    def drain(c, par, sem):
        for r in range(_NIDX):
            pltpu.make_async_copy(table_hbm.at[idx_v.at[r, c]],
                                  rows_v.at[par, r], sem).wait()

    fire(0, 0, sem0)

    def chunk_body(c, carry):
        par = lax.rem(c, 2)
        nxt = c + 1
        npar = lax.rem(nxt, 2)

        @pl.when(jnp.logical_and(nxt < nch, npar == 0))
        def _():
            fire(nxt, 0, sem0)

        @pl.when(jnp.logical_and(nxt < nch, npar == 1))
        def _():
            fire(nxt, 1, sem1)

        @pl.when(par == 0)
        def _():
            drain(c, 0, sem0)

        @pl.when(par == 1)
        def _():
            drain(c, 1, sem1)

        def b_body(b, bc):
            inp = [rows_v[par, 0, b, pl.ds(_L * k, _L)] for k in range(8)]
            for j in range(6):
                acc = inp[0] * rows_v[par, j + 1, b, pl.ds(0, _L)]
                for k in range(1, 8):
                    acc = acc + inp[k] * rows_v[par, j + 1, b, pl.ds(_L * k, _L)]
                acc_v[j, b, :] = acc
            return bc

        lax.fori_loop(0, _CHUNK, b_body, 0, unroll=False)

        def fold_grp(g, gc):
            for j in range(6):
                vs = [acc_v[j, g * _L + i, :] for i in range(_L)]
                res = _fold16(vs, lane, perms)
                if j > 0:
                    res = 0.0 - res
                scores_v[j, pl.ds(c * _CHUNK + g * _L, _L)] = res
            return gc

        lax.fori_loop(0, _CHUNK // _L, fold_grp, 0, unroll=False)
        return carry

    lax.fori_loop(0, nch, chunk_body, 0, unroll=False)
    pltpu.sync_copy(scores_v, out_hbm.at[wid])


def _sc_scores(table, idx, batch, nch):
    mesh = plsc.VectorSubcoreMesh(
        core_axis_name="c", subcore_axis_name="s",
        num_cores=_NC, num_subcores=_NS,
    )
    bpw = nch * _CHUNK
    fn = pl.kernel(
        functools.partial(_sc_body, nch),
        out_type=jax.ShapeDtypeStruct((_NW, 6, bpw), jnp.float32),
        mesh=mesh,
        scratch_types=[
            pltpu.VMEM((_NIDX, nch, _CHUNK), jnp.int32),
            pltpu.VMEM((2, _NIDX, _CHUNK, 128), jnp.float32),
            pltpu.VMEM((6, _CHUNK, _L), jnp.float32),
            pltpu.VMEM((6, bpw), jnp.float32),
            pltpu.SemaphoreType.DMA,
            pltpu.SemaphoreType.DMA,
        ],
    )
    return fn(table, idx)


def _tc_reduce_body(batch, x_ref, o_ref):
    x = x_ref[...]
    ls = jnp.minimum(x, 0.0) - jnp.log(1.0 + jnp.exp(-jnp.abs(x)))
    o_ref[0, 0] = -jnp.sum(ls) / batch


def kernel(input_labels, out_labels, negatives, table):
    batch = input_labels.shape[0]
    assert batch % (_NW * _CHUNK) == 0
    nch = batch // (_NW * _CHUNK)

    il = input_labels.astype(jnp.int32)
    ol = out_labels.astype(jnp.int32)
    ng = negatives.astype(jnp.int32)
    idx7 = jnp.concatenate([il[None, :], ol[None, :], ng.T], axis=0)
    idx = idx7.reshape(_NIDX, _NW, nch, _CHUNK).transpose(1, 0, 2, 3)

    scores = _sc_scores(table.astype(jnp.float32), idx, batch, nch)

    loss = pl.pallas_call(
        functools.partial(_tc_reduce_body, batch),
        out_shape=jax.ShapeDtypeStruct((1, 1), jnp.float32),
        out_specs=pl.BlockSpec(memory_space=pltpu.SMEM),
    )(scores.reshape(batch * 6 // 128, 128))
    return loss[0, 0]


# speedup vs baseline: 3.6815x; 1.3752x over previous
import functools

import jax
import jax.numpy as jnp
from jax import lax
from jax.experimental import pallas as pl
from jax.experimental.pallas import tpu as pltpu
from jax.experimental.pallas import tpu_sc as plsc

_NC = 2
_NS = 16
_NW = _NC * _NS
_L = 16
_NEG = 5
_NIDX = _NEG + 2
_CHUNK = 32


def _lane_shuffle(x, idx):
    return lax.gather(
        x, idx[:, None],
        lax.GatherDimensionNumbers(
            offset_dims=(), collapsed_slice_dims=(0,), start_index_map=(0,)),
        (1,),
        mode=lax.GatherScatterMode.PROMISE_IN_BOUNDS,
    )


def _fold16(vs, lane, perms):
    cur = list(vs)
    for s, p in zip((8, 4, 2, 1), perms):
        h = len(cur) // 2
        nxt = []
        for i in range(h):
            a = cur[i] + _lane_shuffle(cur[i], p)
            b = cur[i + h] + _lane_shuffle(cur[i + h], p)
            nxt.append(jnp.where((lane & s) == 0, a, b))
        cur = nxt
    return cur[0]


def _sc_body(nch, table_hbm, idx_hbm, out_hbm,
             idx_v, rows_v, acc_v, scores_v, sem0, sem1):
    wid = lax.axis_index("s") * _NC + lax.axis_index("c")
    pltpu.sync_copy(idx_hbm.at[wid], idx_v)
    lane = lax.iota(jnp.int32, _L)
    perms = [jnp.bitwise_xor(lane, s) for s in (8, 4, 2, 1)]

    def fire(c, par, sem):
        for r in range(_NIDX):
            pltpu.async_copy(table_hbm.at[idx_v.at[r, c]],
                             rows_v.at[par, r], sem)

    def drain(c, par, sem):
        for r in range(_NIDX):
            pltpu.make_async_copy(table_hbm.at[idx_v.at[r, c]],
                                  rows_v.at[par, r], sem).wait()

    fire(0, 0, sem0)

    def compute(c, par):
        @plsc.parallel_loop(0, _CHUNK, unroll=2)
        def b_body(b):
            inp = [rows_v[par, 0, b, pl.ds(_L * k, _L)] for k in range(8)]
            for j in range(6):
                acc = inp[0] * rows_v[par, j + 1, b, pl.ds(0, _L)]
                for k in range(1, 8):
                    acc = acc + inp[k] * rows_v[par, j + 1, b, pl.ds(_L * k, _L)]
                acc_v[j, b, :] = acc

        @plsc.parallel_loop(0, _CHUNK // _L)
        def fold_grp(g):
            for j in range(6):
                vs = [acc_v[j, g * _L + i, :] for i in range(_L)]
                res = _fold16(vs, lane, perms)
                if j > 0:
                    res = 0.0 - res
                scores_v[j, pl.ds(c * _CHUNK + g * _L, _L)] = res

    def pair_body(c2, carry):
        c = c2 * 2
        fire(c + 1, 1, sem1)
        drain(c, 0, sem0)
        compute(c, 0)

        @pl.when(c + 2 < nch)
        def _():
            fire(c + 2, 0, sem0)

        drain(c + 1, 1, sem1)
        compute(c + 1, 1)
        return carry

    lax.fori_loop(0, nch // 2, pair_body, 0, unroll=False)
    pltpu.sync_copy(scores_v, out_hbm.at[wid])


def _sc_scores(table, idx, batch, nch):
    mesh = plsc.VectorSubcoreMesh(
        core_axis_name="c", subcore_axis_name="s",
        num_cores=_NC, num_subcores=_NS,
    )
    bpw = nch * _CHUNK
    fn = pl.kernel(
        functools.partial(_sc_body, nch),
        out_type=jax.ShapeDtypeStruct((_NW, 6, bpw), jnp.float32),
        mesh=mesh,
        scratch_types=[
            pltpu.VMEM((_NIDX, nch, _CHUNK), jnp.int32),
            pltpu.VMEM((2, _NIDX, _CHUNK, 128), jnp.float32),
            pltpu.VMEM((6, _CHUNK, _L), jnp.float32),
            pltpu.VMEM((6, bpw), jnp.float32),
            pltpu.SemaphoreType.DMA,
            pltpu.SemaphoreType.DMA,
        ],
    )
    return fn(table, idx)


def _tc_reduce_body(batch, x_ref, o_ref):
    x = x_ref[...]
    ls = jnp.minimum(x, 0.0) - jnp.log(1.0 + jnp.exp(-jnp.abs(x)))
    o_ref[0, 0] = -jnp.sum(ls) / batch


def kernel(input_labels, out_labels, negatives, table):
    batch = input_labels.shape[0]
    assert batch % (_NW * _CHUNK) == 0
    nch = batch // (_NW * _CHUNK)

    il = input_labels.astype(jnp.int32)
    ol = out_labels.astype(jnp.int32)
    ng = negatives.astype(jnp.int32)
    idx7 = jnp.concatenate([il[None, :], ol[None, :], ng.T], axis=0)
    idx = idx7.reshape(_NIDX, _NW, nch, _CHUNK).transpose(1, 0, 2, 3)

    scores = _sc_scores(table.astype(jnp.float32), idx, batch, nch)

    loss = pl.pallas_call(
        functools.partial(_tc_reduce_body, batch),
        out_shape=jax.ShapeDtypeStruct((1, 1), jnp.float32),
        out_specs=pl.BlockSpec(memory_space=pltpu.SMEM),
    )(scores.reshape(batch * 6 // 128, 128))
    return loss[0, 0]
